# baseline (speedup 1.0000x reference)
import functools
import jax
import jax.numpy as jnp
from jax import lax
from jax.experimental import pallas as pl
from jax.experimental.pallas import tpu as pltpu
from jax.experimental.pallas import tpu_sc as plsc

NC = 2
NS = 16
NW = NC * NS
LANES = 16
D = 64
CHUNK = 128
NB = 2
NI = 4


@functools.partial(jax.jit, static_argnums=(9, 10, 11, 12))
def _spmm2_sc(ego0, dst0, src0, val0, ego1, dst1, src1, val1, zeros,
              n_pad0, n_chunks0, n_pad1, n_chunks1):
  mesh = plsc.VectorSubcoreMesh(core_axis_name="c", subcore_axis_name="s")

  @functools.partial(
      pl.kernel,
      out_type=[jax.ShapeDtypeStruct((NC, n_pad0, D), jnp.float32),
                jax.ShapeDtypeStruct((NC, n_pad1, D), jnp.float32)],
      mesh=mesh,
      scratch_types=[
          pltpu.VMEM((n_chunks0, CHUNK), jnp.int32),
          pltpu.VMEM((NI, CHUNK), jnp.int32),
          pltpu.VMEM((NI, CHUNK), jnp.float32),
          pltpu.VMEM((NB, CHUNK, D), jnp.float32),
          pltpu.VMEM((NB, CHUNK, D), jnp.float32),
          pltpu.VMEM_SHARED((n_pad0, D), jnp.float32),
          pltpu.VMEM_SHARED((n_pad0, D), jnp.float32),
          [pltpu.SemaphoreType.DMA] * NB,
          [pltpu.SemaphoreType.DMA] * NB,
          [pltpu.SemaphoreType.DMA] * NI,
      ],
      compiler_params=pltpu.CompilerParams(use_tc_tiling_on_sc=False),
  )
  def k(ego0_hbm, dst0_hbm, src0_hbm, val0_hbm, ego1_hbm, dst1_hbm,
        src1_hbm, val1_hbm, zero_hbm, out0_hbm, out1_hbm,
        src_v, dst_r, val_r, rows_v, wrows_v, ego_sh, acc_sh,
        gsems, ssems, isems):
    cid = lax.axis_index("c")
    sid = lax.axis_index("s")
    wid = sid * NC + cid

    def phase(ego_hbm, dst_hbm, src_hbm, val_hbm, out_hbm, n_pad, n_chunks):
      rps = n_pad // NS

      pltpu.sync_copy(ego_hbm.at[pl.ds(sid * rps, rps)],
                      ego_sh.at[pl.ds(sid * rps, rps)])
      pltpu.sync_copy(zero_hbm.at[pl.ds(sid * rps, rps)],
                      acc_sh.at[pl.ds(sid * rps, rps)])
      pltpu.sync_copy(src_hbm.at[wid], src_v.at[pl.ds(0, n_chunks)])
      plsc.subcore_barrier()

      def start_idx(j, s):
        pltpu.async_copy(dst_hbm.at[wid, j], dst_r.at[s], isems[s])
        pltpu.async_copy(val_hbm.at[wid, j], val_r.at[s], isems[s])

      def wait_idx(j, s):
        pltpu.make_async_copy(dst_hbm.at[wid, j], dst_r.at[s],
                              isems[s]).wait()
        pltpu.make_async_copy(val_hbm.at[wid, j], val_r.at[s],
                              isems[s]).wait()

      def start_gather(j, s, b):
        pltpu.async_copy(ego_sh.at[dst_r.at[s]], rows_v.at[b], gsems[b])

      def wait_gather(j, s, b):
        pltpu.make_async_copy(ego_sh.at[dst_r.at[s]], rows_v.at[b],
                              gsems[b]).wait()

      def start_scatter(j, b):
        pltpu.async_copy(wrows_v.at[b], acc_sh.at[src_v.at[j]], ssems[b],
                         add=True)

      def wait_scatter(j, b):
        pltpu.make_async_copy(wrows_v.at[b], acc_sh.at[src_v.at[j]],
                              ssems[b]).wait()

      def multiply(j, s, b):
        def grp_body(g, c2):
          vvec = val_r[s, pl.ds(g * LANES, LANES)]
          for e in range(LANES):
            w = lax.gather(
                vvec, jnp.full((LANES, 1), e, jnp.int32),
                lax.GatherDimensionNumbers(offset_dims=(),
                                           collapsed_slice_dims=(0,),
                                           start_index_map=(0,)),
                (1,), mode=lax.GatherScatterMode.PROMISE_IN_BOUNDS)
            row = g * LANES + e
            for c in range(D // LANES):
              sl = pl.ds(c * LANES, LANES)
              wrows_v[b, row, sl] = rows_v[b, row, sl] * w
          return c2
        lax.fori_loop(0, CHUNK // LANES, grp_body, 0)

      def body(j, s, b, *, idx_ahead=True, gath_ahead=True,
               scat_wait=True):
        if gath_ahead:
          wait_idx(j + 1, (s + 1) % NI)
          start_gather(j + 1, (s + 1) % NI, (b + 1) % NB)
        wait_gather(j, s, b)
        if scat_wait:
          wait_scatter(j - NB, b)
        multiply(j, s, b)
        start_scatter(j, b)
        if idx_ahead:
          start_idx(j + 3, (s + 3) % NI)

      for s in range(3):
        start_idx(s, s)
      wait_idx(0, 0)
      start_gather(0, 0, 0)
      for j in range(4):
        body(j, j % NI, j % NB, scat_wait=(j >= NB))

      def mid(jo, carry):
        for j2 in range(NI):
          j = 4 + jo * NI + j2
          body(j, j2, j2 % NB)
        return carry
      lax.fori_loop(0, (n_chunks - 8) // NI, mid, 0)

      for j in range(n_chunks - 4, n_chunks):
        body(j, j % NI, j % NB, idx_ahead=(j + 3 < n_chunks),
             gath_ahead=(j < n_chunks - 1))
      for j in range(n_chunks - NB, n_chunks):
        wait_scatter(j, j % NB)

      plsc.subcore_barrier()

      pltpu.sync_copy(acc_sh.at[pl.ds(sid * rps, rps)],
                      out_hbm.at[cid, pl.ds(sid * rps, rps)])
      plsc.subcore_barrier()

    phase(ego0_hbm, dst0_hbm, src0_hbm, val0_hbm, out0_hbm,
          n_pad0, n_chunks0)
    phase(ego1_hbm, dst1_hbm, src1_hbm, val1_hbm, out1_hbm,
          n_pad1, n_chunks1)

  return k(ego0, dst0, src0, val0, ego1, dst1, src1, val1, zeros)


def _leaky(x):
  return jnp.where(x >= 0, x, 0.01 * x)


@functools.partial(jax.jit, static_argnums=(6,))
def _layer_tc(part, ego, gw, gb, bw, bb, blk):
  n = ego.shape[0]

  def body(p_ref, e_ref, gw_ref, gb_ref, bw_ref, bb_ref, ne_ref, no_ref):
    side = p_ref[0] + p_ref[1]
    ego_b = e_ref[...]
    sum_e = _leaky(jnp.dot(side, gw_ref[...],
                           preferred_element_type=jnp.float32) + gb_ref[...])
    bi = _leaky(jnp.dot(ego_b * side, bw_ref[...],
                        preferred_element_type=jnp.float32) + bb_ref[...])
    new = sum_e + bi
    nrm = jnp.maximum(
        jnp.sqrt(jnp.sum(new * new, axis=1, keepdims=True)), 1e-12)
    ne_ref[...] = new
    no_ref[...] = new / nrm

  grid = (n // blk,)
  return pl.pallas_call(
      body,
      grid=grid,
      in_specs=[
          pl.BlockSpec((NC, blk, D), lambda i: (0, i, 0)),
          pl.BlockSpec((blk, D), lambda i: (i, 0)),
          pl.BlockSpec((D, D), lambda i: (0, 0)),
          pl.BlockSpec((D,), lambda i: (0,)),
          pl.BlockSpec((D, D), lambda i: (0, 0)),
          pl.BlockSpec((D,), lambda i: (0,)),
      ],
      out_specs=[
          pl.BlockSpec((blk, D), lambda i: (i, 0)),
          pl.BlockSpec((blk, D), lambda i: (i, 0)),
      ],
      out_shape=[
          jax.ShapeDtypeStruct((n, D), jnp.float32),
          jax.ShapeDtypeStruct((n, D), jnp.float32),
      ],
  )(part, ego, gw, gb, bw, bb)


@functools.partial(jax.jit, static_argnums=(3,))
def _addmm_tc(base, w, x, blk):
  m, k = w.shape
  _, n = x.shape

  def body(b_ref, w_ref, x_ref, o_ref):
    o_ref[...] = b_ref[...] + jnp.dot(
        w_ref[...], x_ref[...], preferred_element_type=jnp.float32)

  return pl.pallas_call(
      body,
      grid=(m // blk,),
      in_specs=[
          pl.BlockSpec((blk, n), lambda i: (i, 0)),
          pl.BlockSpec((blk, k), lambda i: (i, 0)),
          pl.BlockSpec((k, n), lambda i: (0, 0)),
      ],
      out_specs=pl.BlockSpec((blk, n), lambda i: (i, 0)),
      out_shape=jax.ShapeDtypeStruct((m, n), jnp.float32),
  )(base, w, x)


def _prep_edges(adj_idx, adj_val, n_chunks):
  e = adj_val.shape[0]
  e_pad = NW * n_chunks * CHUNK
  pad = e_pad - e
  src = jnp.pad(adj_idx[0], (0, pad)).reshape(NW, n_chunks, CHUNK)
  dst = jnp.pad(adj_idx[1], (0, pad)).reshape(NW, n_chunks, CHUNK)
  val = jnp.pad(adj_val, (0, pad)).reshape(NW, n_chunks, CHUNK)
  return dst, src, val


def kernel(adj0_idx, adj0_val, adj1_idx, adj1_val, u_emb0, i_emb0, u_emb1,
           i_emb1, m0_gc_w0, m0_gc_b0, m0_bi_w0, m0_bi_b0, m0_gc_w1, m0_gc_b1,
           m0_bi_w1, m0_bi_b1, m1_gc_w0, m1_gc_b0, m1_bi_w0, m1_bi_b0,
           m1_gc_w1, m1_gc_b1, m1_bi_w1, m1_bi_b1, local_u_w, local_i_w):
  layers0 = [(m0_gc_w0, m0_gc_b0, m0_bi_w0, m0_bi_b0),
             (m0_gc_w1, m0_gc_b1, m0_bi_w1, m0_bi_b1)]
  layers1 = [(m1_gc_w0, m1_gc_b0, m1_bi_w0, m1_bi_b0),
             (m1_gc_w1, m1_gc_b1, m1_bi_w1, m1_bi_b1)]

  np0, nc0, np1, nc1 = 10000, 80, 3200, 24
  ego0 = jnp.concatenate([u_emb0, i_emb0], axis=0)
  ego1 = jnp.pad(jnp.concatenate([u_emb1, i_emb1], axis=0),
                 ((0, np1 - 3000), (0, 0)))
  dst0, src0, val0 = _prep_edges(adj0_idx, adj0_val, nc0)
  dst1, src1, val1 = _prep_edges(adj1_idx, adj1_val, nc1)
  zeros = jnp.zeros((np0, D), jnp.float32)

  outs0, outs1 = [ego0], [ego1]
  for (gw0, gb0, bw0, bb0), (gw1, gb1, bw1, bb1) in zip(layers0, layers1):
    p0, p1 = _spmm2_sc(ego0, dst0, src0, val0, ego1, dst1, src1, val1,
                       zeros, np0, nc0, np1, nc1)
    ego0, n0 = _layer_tc(p0, ego0, gw0, gb0, bw0, bb0, 400)
    ego1, n1 = _layer_tc(p1, ego1, gw1, gb1, bw1, bb1, 400)
    outs0.append(n0)
    outs1.append(n1)
  all0 = jnp.concatenate(outs0, axis=1)
  all1 = jnp.concatenate(outs1, axis=1)[:3000]

  nu0, ni0 = u_emb0.shape[0], i_emb0.shape[0]
  nu1 = u_emb1.shape[0]
  u0, i0 = all0[:nu0], all0[nu0:]
  u1, i1 = all1[:nu1], all1[nu1:]

  user_embd = _addmm_tc(u0, local_u_w, u1, blk=400)
  item_embd = _addmm_tc(i0, local_i_w, i1, blk=400)
  return (user_embd, item_embd)


# speedup vs baseline: 1.2033x; 1.2033x over previous
import functools
import jax
import jax.numpy as jnp
from jax import lax
from jax.experimental import pallas as pl
from jax.experimental.pallas import tpu as pltpu
from jax.experimental.pallas import tpu_sc as plsc

NC = 2
NS = 16
NW = NC * NS
LANES = 16
D = 64
CHUNK = 128
NB = 2
NI = 4


@functools.partial(jax.jit, static_argnums=(5, 6))
def _spmm_sc(ego, dst, src, val, zeros, n_pad, n_chunks):
  rps = n_pad // NS
  mesh = plsc.VectorSubcoreMesh(core_axis_name="c", subcore_axis_name="s")

  @functools.partial(
      pl.kernel,
      out_type=jax.ShapeDtypeStruct((NC, n_pad, D), jnp.float32),
      mesh=mesh,
      scratch_types=[
          pltpu.VMEM((n_chunks, CHUNK), jnp.int32),
          pltpu.VMEM((NI, CHUNK), jnp.int32),
          pltpu.VMEM((NI, CHUNK), jnp.float32),
          pltpu.VMEM((NB, CHUNK, D), jnp.float32),
          pltpu.VMEM((NB, CHUNK, D), jnp.float32),
          pltpu.VMEM_SHARED((n_pad, D), jnp.float32),
          pltpu.VMEM_SHARED((n_pad, D), jnp.float32),
          [pltpu.SemaphoreType.DMA] * NB,
          [pltpu.SemaphoreType.DMA] * NB,
          [pltpu.SemaphoreType.DMA] * NI,
      ],
      compiler_params=pltpu.CompilerParams(use_tc_tiling_on_sc=False),
  )
  def k(ego_hbm, dst_hbm, src_hbm, val_hbm, zero_hbm, out_hbm,
        src_v, dst_r, val_r, rows_v, wrows_v, ego_sh, acc_sh,
        gsems, ssems, isems):
    cid = lax.axis_index("c")
    sid = lax.axis_index("s")
    wid = sid * NC + cid

    pltpu.sync_copy(ego_hbm.at[pl.ds(sid * rps, rps)],
                    ego_sh.at[pl.ds(sid * rps, rps)])
    pltpu.sync_copy(zero_hbm.at[pl.ds(sid * rps, rps)],
                    acc_sh.at[pl.ds(sid * rps, rps)])
    pltpu.sync_copy(src_hbm.at[wid], src_v)
    plsc.subcore_barrier()

    def start_idx(j, s):
      pltpu.async_copy(dst_hbm.at[wid, j], dst_r.at[s], isems[s])
      pltpu.async_copy(val_hbm.at[wid, j], val_r.at[s], isems[s])

    def wait_idx(j, s):
      pltpu.make_async_copy(dst_hbm.at[wid, j], dst_r.at[s],
                            isems[s]).wait()
      pltpu.make_async_copy(val_hbm.at[wid, j], val_r.at[s],
                            isems[s]).wait()

    def start_gather(j, s, b):
      pltpu.async_copy(ego_sh.at[dst_r.at[s]], rows_v.at[b], gsems[b])

    def wait_gather(j, s, b):
      pltpu.make_async_copy(ego_sh.at[dst_r.at[s]], rows_v.at[b],
                            gsems[b]).wait()

    def start_scatter(j, b):
      pltpu.async_copy(wrows_v.at[b], acc_sh.at[src_v.at[j]], ssems[b],
                       add=True)

    def wait_scatter(j, b):
      pltpu.make_async_copy(wrows_v.at[b], acc_sh.at[src_v.at[j]],
                            ssems[b]).wait()

    def multiply(j, s, b):
      def grp_body(g, c2):
        vvec = val_r[s, pl.ds(g * LANES, LANES)]
        for e in range(LANES):
          w = lax.gather(
              vvec, jnp.full((LANES, 1), e, jnp.int32),
              lax.GatherDimensionNumbers(offset_dims=(),
                                         collapsed_slice_dims=(0,),
                                         start_index_map=(0,)),
              (1,), mode=lax.GatherScatterMode.PROMISE_IN_BOUNDS)
          row = g * LANES + e
          for c in range(D // LANES):
            sl = pl.ds(c * LANES, LANES)
            wrows_v[b, row, sl] = rows_v[b, row, sl] * w
        return c2
      lax.fori_loop(0, CHUNK // LANES, grp_body, 0)

    def body(j, s, b, *, idx_ahead=True, gath_ahead=True, scat_wait=True):
      if gath_ahead:
        wait_idx(j + 1, (s + 1) % NI)
        start_gather(j + 1, (s + 1) % NI, (b + 1) % NB)
      wait_gather(j, s, b)
      if scat_wait:
        wait_scatter(j - NB, b)
      multiply(j, s, b)
      start_scatter(j, b)
      if idx_ahead:
        start_idx(j + 3, (s + 3) % NI)

    for s in range(3):
      start_idx(s, s)
    wait_idx(0, 0)
    start_gather(0, 0, 0)
    for j in range(4):
      body(j, j % NI, j % NB, scat_wait=(j >= NB))

    def mid(jo, carry):
      for j2 in range(NI):
        j = 4 + jo * NI + j2
        body(j, j2, j2 % NB)
      return carry
    lax.fori_loop(0, (n_chunks - 8) // NI, mid, 0)

    for j in range(n_chunks - 4, n_chunks):
      body(j, j % NI, j % NB, idx_ahead=(j + 3 < n_chunks),
           gath_ahead=(j < n_chunks - 1))
    for j in range(n_chunks - NB, n_chunks):
      wait_scatter(j, j % NB)

    plsc.subcore_barrier()

    pltpu.sync_copy(acc_sh.at[pl.ds(sid * rps, rps)],
                    out_hbm.at[cid, pl.ds(sid * rps, rps)])

  return k(ego, dst, src, val, zeros)


def _leaky(x):
  return jnp.where(x >= 0, x, 0.01 * x)


@functools.partial(jax.jit, static_argnums=(6,))
def _layer_tc(part, ego, gw, gb, bw, bb, blk):
  n = ego.shape[0]

  def body(p_ref, e_ref, gw_ref, gb_ref, bw_ref, bb_ref, ne_ref, no_ref):
    side = p_ref[0] + p_ref[1]
    ego_b = e_ref[...]
    sum_e = _leaky(jnp.dot(side, gw_ref[...],
                           preferred_element_type=jnp.float32) + gb_ref[...])
    bi = _leaky(jnp.dot(ego_b * side, bw_ref[...],
                        preferred_element_type=jnp.float32) + bb_ref[...])
    new = sum_e + bi
    nrm = jnp.maximum(
        jnp.sqrt(jnp.sum(new * new, axis=1, keepdims=True)), 1e-12)
    ne_ref[...] = new
    no_ref[...] = new / nrm

  grid = (n // blk,)
  return pl.pallas_call(
      body,
      grid=grid,
      in_specs=[
          pl.BlockSpec((NC, blk, D), lambda i: (0, i, 0)),
          pl.BlockSpec((blk, D), lambda i: (i, 0)),
          pl.BlockSpec((D, D), lambda i: (0, 0)),
          pl.BlockSpec((D,), lambda i: (0,)),
          pl.BlockSpec((D, D), lambda i: (0, 0)),
          pl.BlockSpec((D,), lambda i: (0,)),
      ],
      out_specs=[
          pl.BlockSpec((blk, D), lambda i: (i, 0)),
          pl.BlockSpec((blk, D), lambda i: (i, 0)),
      ],
      out_shape=[
          jax.ShapeDtypeStruct((n, D), jnp.float32),
          jax.ShapeDtypeStruct((n, D), jnp.float32),
      ],
  )(part, ego, gw, gb, bw, bb)


@functools.partial(jax.jit, static_argnums=(3,))
def _addmm_tc(base, w, x, blk):
  m, k = w.shape
  _, n = x.shape

  def body(b_ref, w_ref, x_ref, o_ref):
    o_ref[...] = b_ref[...] + jnp.dot(
        w_ref[...], x_ref[...], preferred_element_type=jnp.float32)

  return pl.pallas_call(
      body,
      grid=(m // blk,),
      in_specs=[
          pl.BlockSpec((blk, n), lambda i: (i, 0)),
          pl.BlockSpec((blk, k), lambda i: (i, 0)),
          pl.BlockSpec((k, n), lambda i: (0, 0)),
      ],
      out_specs=pl.BlockSpec((blk, n), lambda i: (i, 0)),
      out_shape=jax.ShapeDtypeStruct((m, n), jnp.float32),
  )(base, w, x)


def _prep_edges(adj_idx, adj_val, n_chunks):
  e = adj_val.shape[0]
  e_pad = NW * n_chunks * CHUNK
  pad = e_pad - e
  src = jnp.pad(adj_idx[0], (0, pad)).reshape(NW, n_chunks, CHUNK)
  dst = jnp.pad(adj_idx[1], (0, pad)).reshape(NW, n_chunks, CHUNK)
  val = jnp.pad(adj_val, (0, pad)).reshape(NW, n_chunks, CHUNK)
  return dst, src, val


def _ngcf_model(adj_idx, adj_val, u_emb, i_emb, layers, n_pad, n_chunks, blk):
  n_real = u_emb.shape[0] + i_emb.shape[0]
  ego = jnp.concatenate([u_emb, i_emb], axis=0)
  if n_pad != n_real:
    ego = jnp.pad(ego, ((0, n_pad - n_real), (0, 0)))
  dst, src, val = _prep_edges(adj_idx, adj_val, n_chunks)
  zeros = jnp.zeros((n_pad, D), jnp.float32)
  outs = [ego]
  for gw, gb, bw, bb in layers:
    part = _spmm_sc(ego, dst, src, val, zeros, n_pad, n_chunks)
    ego, normed = _layer_tc(part, ego, gw, gb, bw, bb, blk)
    outs.append(normed)
  all_e = jnp.concatenate(outs, axis=1)
  return all_e[:n_real]


def kernel(adj0_idx, adj0_val, adj1_idx, adj1_val, u_emb0, i_emb0, u_emb1,
           i_emb1, m0_gc_w0, m0_gc_b0, m0_bi_w0, m0_bi_b0, m0_gc_w1, m0_gc_b1,
           m0_bi_w1, m0_bi_b1, m1_gc_w0, m1_gc_b0, m1_bi_w0, m1_bi_b0,
           m1_gc_w1, m1_gc_b1, m1_bi_w1, m1_bi_b1, local_u_w, local_i_w):
  layers0 = [(m0_gc_w0, m0_gc_b0, m0_bi_w0, m0_bi_b0),
             (m0_gc_w1, m0_gc_b1, m0_bi_w1, m0_bi_b1)]
  layers1 = [(m1_gc_w0, m1_gc_b0, m1_bi_w0, m1_bi_b0),
             (m1_gc_w1, m1_gc_b1, m1_bi_w1, m1_bi_b1)]

  all0 = _ngcf_model(adj0_idx, adj0_val, u_emb0, i_emb0, layers0,
                     n_pad=10000, n_chunks=80, blk=400)
  all1 = _ngcf_model(adj1_idx, adj1_val, u_emb1, i_emb1, layers1,
                     n_pad=3200, n_chunks=24, blk=400)

  nu0, ni0 = u_emb0.shape[0], i_emb0.shape[0]
  nu1 = u_emb1.shape[0]
  u0, i0 = all0[:nu0], all0[nu0:]
  u1, i1 = all1[:nu1], all1[nu1:]

  user_embd = _addmm_tc(u0, local_u_w, u1, blk=400)
  item_embd = _addmm_tc(i0, local_i_w, i1, blk=400)
  return (user_embd, item_embd)
